# Initial kernel scaffold; baseline (speedup 1.0000x reference)
import jax
import jax.numpy as jnp
from jax.experimental import pallas as pl


def kernel(x, edge_index, W1_l, b1, W1_r, W2_l, b2, W2_r, W3_l, b3, W3_r, Wf, bf):
    raise NotImplementedError("write your pallas kernel here")


# speedup vs baseline: 4.5771x; 4.5771x over previous
import functools

import jax
import jax.numpy as jnp
from jax import lax
from jax.experimental import pallas as pl
from jax.experimental.pallas import tpu as pltpu
from jax.experimental.pallas import tpu_sc as plsc

N_NODES = 10000
N_EDGES = 320000
D = 64

NW = 32
BLK = 128
EPW = 10240
E_PAD = NW * EPW
NBLK = EPW // BLK
NACC = 10240
RPS = NACC // 16


def _seg_sum_body(with_cnt, *refs):
    if with_cnt:
        (p_hbm, src_hbm, dst_hbm, z64_hbm, z16_hbm, ones_hbm,
         out_hbm, cnt_hbm,
         src_v, dst_v, rows_v, ones_v, acc_sh, cnt_sh, sem) = refs
    else:
        (p_hbm, src_hbm, dst_hbm, z64_hbm,
         out_hbm,
         src_v, dst_v, rows_v, acc_sh, sem) = refs

    c = lax.axis_index("c")
    s = lax.axis_index("s")
    wid = s * 2 + c

    pltpu.sync_copy(z64_hbm, acc_sh.at[pl.ds(s * RPS, RPS)])
    if with_cnt:
        pltpu.sync_copy(z16_hbm, cnt_sh.at[pl.ds(s * RPS, RPS)])
        pltpu.sync_copy(ones_hbm, ones_v)
    plsc.subcore_barrier()

    base = wid * EPW

    def blk(j, carry):
        off = base + j * BLK
        pltpu.sync_copy(src_hbm.at[pl.ds(off, BLK)], src_v)
        pltpu.sync_copy(dst_hbm.at[pl.ds(off, BLK)], dst_v)
        pltpu.async_copy(p_hbm.at[src_v], rows_v, sem).wait()
        pltpu.sync_copy(rows_v, acc_sh.at[dst_v], add=True)
        if with_cnt:
            pltpu.sync_copy(ones_v, cnt_sh.at[dst_v], add=True)
        return carry

    lax.fori_loop(0, NBLK, blk, 0)
    plsc.subcore_barrier()

    pltpu.sync_copy(acc_sh.at[pl.ds(s * RPS, RPS)],
                    out_hbm.at[c, pl.ds(s * RPS, RPS)])
    if with_cnt:
        pltpu.sync_copy(cnt_sh.at[pl.ds(s * RPS, RPS)],
                        cnt_hbm.at[c, pl.ds(s * RPS, RPS)])


def _make_seg_sum(with_cnt):
    mesh = plsc.VectorSubcoreMesh(core_axis_name="c", subcore_axis_name="s")
    out_type = [jax.ShapeDtypeStruct((2, NACC, D), jnp.float32)]
    scratch = [
        pltpu.VMEM((BLK,), jnp.int32),
        pltpu.VMEM((BLK,), jnp.int32),
        pltpu.VMEM((BLK, D), jnp.float32),
    ]
    if with_cnt:
        out_type.append(jax.ShapeDtypeStruct((2, NACC, 16), jnp.float32))
        scratch.append(pltpu.VMEM((BLK, 16), jnp.float32))
        scratch.append(pltpu.VMEM_SHARED((NACC, D), jnp.float32))
        scratch.append(pltpu.VMEM_SHARED((NACC, 16), jnp.float32))
    else:
        scratch.append(pltpu.VMEM_SHARED((NACC, D), jnp.float32))
    scratch.append(pltpu.SemaphoreType.DMA)
    return pl.kernel(
        functools.partial(_seg_sum_body, with_cnt),
        out_type=out_type,
        mesh=mesh,
        scratch_types=scratch,
        compiler_params=pltpu.CompilerParams(use_tc_tiling_on_sc=False),
    )


_seg_sum_cnt = _make_seg_sum(True)
_seg_sum = _make_seg_sum(False)


_R = 2000
_GRID = N_NODES // _R


def _rows(bs):
    return pl.BlockSpec(bs, lambda i: (i, 0))


def _full(bs):
    return pl.BlockSpec(bs, lambda i: (0, 0))


def _tc0_body(x_ref, wl_ref, wr_ref, p_ref, r_ref):
    x = x_ref[...]
    p_ref[...] = jnp.dot(x, wl_ref[...], preferred_element_type=jnp.float32)
    r_ref[...] = jnp.dot(x, wr_ref[...], preferred_element_type=jnp.float32)


_tc0 = pl.pallas_call(
    _tc0_body,
    grid=(_GRID,),
    in_specs=[_rows((_R, 128)), _full((128, D)), _full((128, D))],
    out_specs=[_rows((_R, D)), _rows((_R, D))],
    out_shape=[jax.ShapeDtypeStruct((N_NODES, D), jnp.float32)] * 2,
)


def _tc1_body(pa_ref, pb_ref, cnt_ref, b_ref, r_ref, w2r_ref,
              h_ref, r2_ref):
    inv = 1.0 / jnp.maximum(cnt_ref[...][:, :1], 1.0)
    h = jnp.tanh((pa_ref[...] + pb_ref[...]) * inv + b_ref[...] + r_ref[...])
    h_ref[...] = h
    r2_ref[...] = jnp.dot(h, w2r_ref[...], preferred_element_type=jnp.float32)


_tc1 = pl.pallas_call(
    _tc1_body,
    grid=(_GRID,),
    in_specs=[_rows((_R, D)), _rows((_R, D)), _rows((_R, 16)),
              _full((1, D)), _rows((_R, D)), _full((D, 128))],
    out_specs=[_rows((_R, D)), _rows((_R, 128))],
    out_shape=[jax.ShapeDtypeStruct((N_NODES, D), jnp.float32),
               jax.ShapeDtypeStruct((N_NODES, 128), jnp.float32)],
)


def _tc2_body(pa_ref, pb_ref, cnt_ref, w2l_ref, b_ref, r2_ref,
              w3l_ref, w3r_ref, p3_ref, r3_ref):
    inv = 1.0 / jnp.maximum(cnt_ref[...][:, :1], 1.0)
    agg = (pa_ref[...] + pb_ref[...]) * inv
    h2 = jnp.tanh(jnp.dot(agg, w2l_ref[...], preferred_element_type=jnp.float32)
                  + b_ref[...] + r2_ref[...])
    p3_ref[...] = jnp.dot(h2, w3l_ref[...], preferred_element_type=jnp.float32)
    r3_ref[...] = jnp.dot(h2, w3r_ref[...], preferred_element_type=jnp.float32)


_tc2 = pl.pallas_call(
    _tc2_body,
    grid=(_GRID,),
    in_specs=[_rows((_R, D)), _rows((_R, D)), _rows((_R, 16)),
              _full((D, 128)), _full((1, 128)), _rows((_R, 128)),
              _full((128, D)), _full((128, D))],
    out_specs=[_rows((_R, D)), _rows((_R, D))],
    out_shape=[jax.ShapeDtypeStruct((N_NODES, D), jnp.float32)] * 2,
)


def _tc3_body(pa_ref, pb_ref, cnt_ref, b_ref, r3_ref, wf_ref, bf_ref,
              out_ref):
    inv = 1.0 / jnp.maximum(cnt_ref[...][:, :1], 1.0)
    h3 = jnp.tanh((pa_ref[...] + pb_ref[...]) * inv + b_ref[...] + r3_ref[...])
    out_ref[...] = (jnp.dot(h3, wf_ref[...], preferred_element_type=jnp.float32)
                    + bf_ref[...])


_tc3 = pl.pallas_call(
    _tc3_body,
    grid=(_GRID,),
    in_specs=[_rows((_R, D)), _rows((_R, D)), _rows((_R, 16)),
              _full((1, D)), _rows((_R, D)), _full((D, 40)), _full((1, 40))],
    out_specs=_rows((_R, 40)),
    out_shape=jax.ShapeDtypeStruct((N_NODES, 40), jnp.float32),
)


@jax.jit
def kernel(x, edge_index, W1_l, b1, W1_r, W2_l, b2, W2_r, W3_l, b3, W3_r,
           Wf, bf):
    pad = E_PAD - N_EDGES
    src = jnp.concatenate(
        [edge_index[0].astype(jnp.int32), jnp.zeros((pad,), jnp.int32)])
    dst = jnp.concatenate(
        [edge_index[1].astype(jnp.int32),
         jnp.full((pad,), N_NODES, jnp.int32)])

    z64 = jnp.zeros((RPS, D), jnp.float32)
    z16 = jnp.zeros((RPS, 16), jnp.float32)
    ones = jnp.ones((BLK, 16), jnp.float32)

    p1, r1 = _tc0(x, W1_l, W1_r)
    (agg1, cnt2) = _seg_sum_cnt(p1, src, dst, z64, z16, ones)
    cnt = cnt2[0, :N_NODES, :] + cnt2[1, :N_NODES, :]
    h1, r2 = _tc1(agg1[0, :N_NODES], agg1[1, :N_NODES], cnt,
                  b1.reshape(1, D), r1, W2_r)

    (agg2,) = _seg_sum(h1, src, dst, z64)
    p3, r3 = _tc2(agg2[0, :N_NODES], agg2[1, :N_NODES], cnt,
                  W2_l, b2.reshape(1, 128), r2, W3_l, W3_r)

    (agg3,) = _seg_sum(p3, src, dst, z64)
    out = _tc3(agg3[0, :N_NODES], agg3[1, :N_NODES], cnt,
               b3.reshape(1, D), r3, Wf, bf.reshape(1, 40))
    return out
